# baseline (speedup 1.0000x reference)
import functools

import jax
import jax.numpy as jnp
from jax import lax
from jax.experimental import pallas as pl
from jax.experimental.pallas import tpu as pltpu
from jax.experimental.pallas import tpu_sc as plsc

NX = 432
NY = 496
B = 4
P_PER = 10000
C_FEAT = 64
C_ALL = C_FEAT + 3
P_PAD = P_PER + 16
COLS = 56
BAND = COLS * NY
SENT = P_PER

_mesh = plsc.VectorSubcoreMesh(core_axis_name="c", subcore_axis_name="s")


@functools.partial(
    pl.kernel,
    mesh=_mesh,
    out_type=[
        jax.ShapeDtypeStruct((B, C_FEAT, NX, NY), jnp.float32),
        jax.ShapeDtypeStruct((B, 3, NX, NY), jnp.float32),
    ],
    scratch_types=[
        pltpu.VMEM((P_PAD,), jnp.int32),
        pltpu.VMEM((BAND,), jnp.int32),
        pltpu.VMEM((P_PAD,), jnp.float32),
        pltpu.VMEM((P_PAD,), jnp.float32),
        pltpu.VMEM((COLS, NY), jnp.float32),
        pltpu.VMEM((COLS, NY), jnp.float32),
        pltpu.SemaphoreType.DMA,
        pltpu.SemaphoreType.DMA,
        pltpu.SemaphoreType.DMA,
        pltpu.SemaphoreType.DMA,
        pltpu.SemaphoreType.DMA,
    ],
    compiler_params=pltpu.CompilerParams(needs_layout_passes=False),
)
def _sc_scatter(table_hbm, keys_hbm, feat_hbm, coord_hbm,
                kbuf, wmap, tbuf0, tbuf1, obuf0, obuf1,
                tsem0, tsem1, osem0, osem1, ksem):
    cid = lax.axis_index("c")
    sid = lax.axis_index("s")
    wid = sid * 2 + cid
    b = wid // 8
    sh = wid % 8
    x0 = jnp.minimum(sh * COLS, NX - COLS)
    lo = x0 * NY

    pltpu.async_copy(keys_hbm.at[pl.ds(b * P_PAD, P_PAD)], kbuf, ksem)

    lanes = lax.broadcasted_iota(jnp.int32, (16,), 0)

    @plsc.parallel_loop(0, BAND // 16, unroll=8)
    def _(i):
        wmap[pl.ds(i * 16, 16)] = jnp.full((16,), SENT, jnp.int32)

    pltpu.make_async_copy(keys_hbm.at[pl.ds(b * P_PAD, P_PAD)], kbuf, ksem).wait()

    def p1_body(i, carry):
        idx = kbuf[pl.ds(i * 16, 16)]
        _, keep = plsc.scan_count(idx)
        memb = jnp.logical_and(idx >= lo, idx < lo + BAND)
        mask = jnp.logical_and(keep, memb)
        li = jnp.clip(idx - lo, 0, BAND - 1)
        q = i * 16 + lanes
        plsc.store_scatter(wmap, [li], q, mask=mask)
        return carry

    lax.fori_loop(0, P_PAD // 16, p1_body, 0)

    def tsrc(c):
        return table_hbm.at[pl.ds((b * C_ALL + c) * P_PAD, P_PAD)]

    def fdst(c):
        return feat_hbm.at[b, c, pl.ds(x0, COLS), :]

    def cdst(j):
        return coord_hbm.at[b, j, pl.ds(x0, COLS), :]

    def gather_into(out_buf, table_buf):
        @plsc.parallel_loop(0, COLS, unroll=4)
        def _(r):
            for j in range(NY // 16):
                wv = wmap[pl.ds(r * NY + j * 16, 16)]
                out_buf[r, pl.ds(j * 16, 16)] = plsc.load_gather(table_buf, [wv])

    pltpu.async_copy(tsrc(0), tbuf0, tsem0)
    pltpu.async_copy(tsrc(1), tbuf1, tsem1)

    def chan_body(k, carry):
        c0 = 2 * k

        pltpu.make_async_copy(tsrc(c0), tbuf0, tsem0).wait()

        @pl.when(k > 0)
        def _():
            pltpu.make_async_copy(obuf0, fdst(c0 - 2), osem0).wait()

        gather_into(obuf0, tbuf0)
        pltpu.async_copy(obuf0, fdst(c0), osem0)
        pltpu.async_copy(tsrc(c0 + 2), tbuf0, tsem0)

        pltpu.make_async_copy(tsrc(c0 + 1), tbuf1, tsem1).wait()

        @pl.when(k > 0)
        def _():
            pltpu.make_async_copy(obuf1, fdst(c0 - 1), osem1).wait()

        gather_into(obuf1, tbuf1)
        pltpu.async_copy(obuf1, fdst(c0 + 1), osem1)
        pltpu.async_copy(tsrc(c0 + 3), tbuf1, tsem1)
        return carry

    lax.fori_loop(0, C_FEAT // 2, chan_body, 0)

    pltpu.make_async_copy(tsrc(C_FEAT), tbuf0, tsem0).wait()
    pltpu.make_async_copy(obuf0, fdst(C_FEAT - 2), osem0).wait()
    gather_into(obuf0, tbuf0)
    pltpu.async_copy(obuf0, cdst(0), osem0)

    pltpu.make_async_copy(tsrc(C_FEAT + 1), tbuf1, tsem1).wait()
    pltpu.make_async_copy(obuf1, fdst(C_FEAT - 1), osem1).wait()
    gather_into(obuf1, tbuf1)
    pltpu.async_copy(obuf1, cdst(1), osem1)

    pltpu.make_async_copy(obuf0, cdst(0), osem0).wait()

    @plsc.parallel_loop(0, COLS, unroll=2)
    def _(r):
        for j in range(NY // 16):
            obuf0[r, pl.ds(j * 16, 16)] = jnp.zeros((16,), jnp.float32)

    pltpu.async_copy(obuf0, cdst(2), osem0)

    pltpu.make_async_copy(obuf1, cdst(1), osem1).wait()
    pltpu.make_async_copy(obuf0, cdst(2), osem0).wait()


def kernel(pillar_features, voxel_coords):
    pfb = pillar_features.reshape(B, P_PER, C_FEAT)
    vcb = voxel_coords.reshape(B, P_PER, 4)
    z = vcb[..., 1]
    y = vcb[..., 2]
    x = vcb[..., 3]

    ftb = jnp.swapaxes(pfb, 1, 2)
    coord = jnp.stack([y, x, z], axis=1).astype(jnp.float32)
    table = jnp.concatenate([ftb, coord], axis=1)
    table = jnp.pad(table, ((0, 0), (0, 0), (0, P_PAD - P_PER)))

    idx = x * NY + y + z
    keys = jnp.pad(idx, ((0, 0), (0, P_PAD - P_PER)),
                   constant_values=jnp.int32(NX * NY))

    feat, coord_out = _sc_scatter(table.reshape(-1), keys.reshape(-1))
    return (jnp.swapaxes(feat, 2, 3), jnp.swapaxes(coord_out, 2, 3))


# speedup vs baseline: 1.0788x; 1.0788x over previous
import functools

import jax
import jax.numpy as jnp
from jax import lax
from jax.experimental import pallas as pl
from jax.experimental.pallas import tpu as pltpu
from jax.experimental.pallas import tpu_sc as plsc

NX = 432
NY = 496
B = 4
P_PER = 10000
C_FEAT = 64
C_ALL = C_FEAT + 3
P_PAD = P_PER + 16
COLS = 56
BAND = COLS * NY
SENT = P_PER

_mesh = plsc.VectorSubcoreMesh(core_axis_name="c", subcore_axis_name="s")


@functools.partial(
    pl.kernel,
    mesh=_mesh,
    out_type=[
        jax.ShapeDtypeStruct((B, C_FEAT, NX, NY), jnp.float32),
        jax.ShapeDtypeStruct((B, 3, NX, NY), jnp.float32),
    ],
    scratch_types=[
        pltpu.VMEM((P_PAD,), jnp.int32),
        pltpu.VMEM((BAND,), jnp.int32),
        pltpu.VMEM((P_PAD,), jnp.float32),
        pltpu.VMEM((P_PAD,), jnp.float32),
        pltpu.VMEM((COLS, NY), jnp.float32),
        pltpu.VMEM((COLS, NY), jnp.float32),
        pltpu.SemaphoreType.DMA,
        pltpu.SemaphoreType.DMA,
        pltpu.SemaphoreType.DMA,
        pltpu.SemaphoreType.DMA,
        pltpu.SemaphoreType.DMA,
    ],
    compiler_params=pltpu.CompilerParams(needs_layout_passes=False),
)
def _sc_scatter(table_hbm, keys_hbm, feat_hbm, coord_hbm,
                kbuf, wmap, tbuf0, tbuf1, obuf0, obuf1,
                tsem0, tsem1, osem0, osem1, ksem):
    cid = lax.axis_index("c")
    sid = lax.axis_index("s")
    wid = sid * 2 + cid
    b = wid // 8
    sh = wid % 8
    x0 = jnp.minimum(sh * COLS, NX - COLS)
    lo = x0 * NY

    pltpu.async_copy(keys_hbm.at[pl.ds(b * P_PAD, P_PAD)], kbuf, ksem)

    lanes = lax.broadcasted_iota(jnp.int32, (16,), 0)

    @plsc.parallel_loop(0, BAND // 16, unroll=8)
    def _(i):
        wmap[pl.ds(i * 16, 16)] = jnp.full((16,), SENT, jnp.int32)

    pltpu.make_async_copy(keys_hbm.at[pl.ds(b * P_PAD, P_PAD)], kbuf, ksem).wait()

    def p1_body(i, carry):
        idx = kbuf[pl.ds(i * 16, 16)]
        _, keep = plsc.scan_count(idx)
        memb = jnp.logical_and(idx >= lo, idx < lo + BAND)
        mask = jnp.logical_and(keep, memb)
        li = jnp.clip(idx - lo, 0, BAND - 1)
        q = i * 16 + lanes
        plsc.store_scatter(wmap, [li], q, mask=mask)
        return carry

    lax.fori_loop(0, P_PAD // 16, p1_body, 0)

    def tsrc(c):
        return table_hbm.at[pl.ds((b * C_ALL + c) * P_PAD, P_PAD)]

    def fdst(c):
        return feat_hbm.at[b, c, pl.ds(x0, COLS), :]

    def cdst(j):
        return coord_hbm.at[b, j, pl.ds(x0, COLS), :]

    def gather_into(out_buf, table_buf):
        @plsc.parallel_loop(0, COLS, unroll=2)
        def _(r):
            for j in range(NY // 16):
                wv = wmap[pl.ds(r * NY + j * 16, 16)]
                out_buf[r, pl.ds(j * 16, 16)] = plsc.load_gather(table_buf, [wv])

    pltpu.async_copy(tsrc(0), tbuf0, tsem0)
    pltpu.async_copy(tsrc(1), tbuf1, tsem1)

    def chan_body(k, carry):
        c0 = 2 * k

        pltpu.make_async_copy(tsrc(c0), tbuf0, tsem0).wait()

        @pl.when(k > 0)
        def _():
            pltpu.make_async_copy(obuf0, fdst(c0 - 2), osem0).wait()

        gather_into(obuf0, tbuf0)
        pltpu.async_copy(obuf0, fdst(c0), osem0)
        pltpu.async_copy(tsrc(c0 + 2), tbuf0, tsem0)

        pltpu.make_async_copy(tsrc(c0 + 1), tbuf1, tsem1).wait()

        @pl.when(k > 0)
        def _():
            pltpu.make_async_copy(obuf1, fdst(c0 - 1), osem1).wait()

        gather_into(obuf1, tbuf1)
        pltpu.async_copy(obuf1, fdst(c0 + 1), osem1)
        pltpu.async_copy(tsrc(c0 + 3), tbuf1, tsem1)
        return carry

    lax.fori_loop(0, C_FEAT // 2, chan_body, 0)

    pltpu.make_async_copy(tsrc(C_FEAT), tbuf0, tsem0).wait()
    pltpu.make_async_copy(obuf0, fdst(C_FEAT - 2), osem0).wait()
    gather_into(obuf0, tbuf0)
    pltpu.async_copy(obuf0, cdst(0), osem0)

    pltpu.make_async_copy(tsrc(C_FEAT + 1), tbuf1, tsem1).wait()
    pltpu.make_async_copy(obuf1, fdst(C_FEAT - 1), osem1).wait()
    gather_into(obuf1, tbuf1)
    pltpu.async_copy(obuf1, cdst(1), osem1)

    pltpu.make_async_copy(obuf0, cdst(0), osem0).wait()

    @plsc.parallel_loop(0, COLS, unroll=2)
    def _(r):
        for j in range(NY // 16):
            obuf0[r, pl.ds(j * 16, 16)] = jnp.zeros((16,), jnp.float32)

    pltpu.async_copy(obuf0, cdst(2), osem0)

    pltpu.make_async_copy(obuf1, cdst(1), osem1).wait()
    pltpu.make_async_copy(obuf0, cdst(2), osem0).wait()


def kernel(pillar_features, voxel_coords):
    pfb = pillar_features.reshape(B, P_PER, C_FEAT)
    vcb = voxel_coords.reshape(B, P_PER, 4)
    z = vcb[..., 1]
    y = vcb[..., 2]
    x = vcb[..., 3]

    ftb = jnp.swapaxes(pfb, 1, 2)
    coord = jnp.stack([y, x, z], axis=1).astype(jnp.float32)
    table = jnp.concatenate([ftb, coord], axis=1)
    table = jnp.pad(table, ((0, 0), (0, 0), (0, P_PAD - P_PER)))

    idx = x * NY + y + z
    keys = jnp.pad(idx, ((0, 0), (0, P_PAD - P_PER)),
                   constant_values=jnp.int32(NX * NY))

    feat, coord_out = _sc_scatter(table.reshape(-1), keys.reshape(-1))
    return (jnp.swapaxes(feat, 2, 3), jnp.swapaxes(coord_out, 2, 3))
